# baseline (speedup 1.0000x reference)
import functools
import math

import jax
import jax.numpy as jnp
from jax import lax
from jax.experimental import pallas as pl
from jax.experimental.pallas import tpu as pltpu
from jax.experimental.pallas import tpu_sc as plsc

D_EMBED = 64
SCALE = math.sqrt(D_EMBED)

NC, NS = 2, 16
NW = NC * NS
XRC = 4


def _make_kernel(R, S, RK, r0):
    assert RK % (NW * XRC) == 0
    xr_per_w = RK // NW
    n_chunks = xr_per_w // XRC
    assert n_chunks % 2 == 0
    cs = XRC * S
    b_per_w = xr_per_w * S
    splits = []
    o = 0
    while o < cs:
        n = min(128, cs - o)
        splits.append((o, n))
        o += n
    mesh = plsc.VectorSubcoreMesh(
        core_axis_name="c", subcore_axis_name="s",
        num_cores=NC, num_subcores=NS)

    @functools.partial(
        pl.kernel,
        out_type=jax.ShapeDtypeStruct((RK, S, D_EMBED), jnp.float32),
        mesh=mesh,
        scratch_types=[
            pltpu.VMEM((b_per_w,), jnp.int32),
            pltpu.VMEM((cs, D_EMBED), jnp.float32),
            pltpu.VMEM((cs, D_EMBED), jnp.float32),
            pltpu.VMEM((XRC, S, D_EMBED), jnp.float32),
            pltpu.VMEM((XRC, S, D_EMBED), jnp.float32),
            pltpu.SemaphoreType.DMA((2,)),
            pltpu.SemaphoreType.DMA((2,)),
        ],
        compiler_params=pltpu.CompilerParams(use_tc_tiling_on_sc=False),
    )
    def k(x_hbm, table_hbm, out_hbm, idx_v, gb0, gb1, ob0, ob1, gsem, ssem):
        wid = lax.axis_index("s") * NC + lax.axis_index("c")
        pltpu.sync_copy(
            x_hbm.at[pl.ds(r0 * S + wid * b_per_w, b_per_w)], idx_v)
        gbufs = (gb0, gb1)
        obufs = (ob0, ob1)

        def gather_start(g, b):
            for (o, n) in splits:
                pltpu.async_copy(
                    table_hbm.at[idx_v.at[pl.ds(g * cs + o, n)]],
                    gbufs[b].at[pl.ds(o, n)], gsem.at[b])

        def gather_wait(b):
            for (o, n) in splits:
                pltpu.make_async_copy(
                    table_hbm.at[idx_v.at[pl.ds(o, n)]],
                    gbufs[b].at[pl.ds(o, n)], gsem.at[b]).wait()

        def store_start(g, b):
            pltpu.async_copy(
                obufs[b], out_hbm.at[pl.ds(wid * xr_per_w + XRC * g, XRC)],
                ssem.at[b])

        def store_wait(b):
            pltpu.make_async_copy(
                obufs[b], out_hbm.at[pl.ds(0, XRC)], ssem.at[b]).wait()

        def scale_out(b):
            gb, ob = gbufs[b], obufs[b]

            def body(s, c):
                for r in range(XRC):
                    li = r * S + s
                    for p in range(D_EMBED // 16):
                        ob[r, s, pl.ds(p * 16, 16)] = (
                            gb[li, pl.ds(p * 16, 16)] * SCALE)
                return c
            lax.fori_loop(0, S, body, 0)

        gather_start(0, 0)

        def pair(ti, c):
            for ph in range(2):
                g = 2 * ti + ph
                b, nb = ph, 1 - ph

                @pl.when(jnp.logical_and(g >= 1, g + 1 < n_chunks))
                def _():
                    store_wait(nb)

                @pl.when(g + 1 < n_chunks)
                def _():
                    gather_start(g + 1, nb)

                gather_wait(b)
                scale_out(b)
                store_start(g, b)
            return c
        lax.fori_loop(0, n_chunks // 2, pair, 0)
        store_wait(0)
        store_wait(1)

    return k


def kernel(x, table):
    R, S = x.shape
    x1d = x.reshape(R * S).astype(jnp.int32)
    halves = [
        _make_kernel(R, S, R // 2, r0)(x1d, table)
        for r0 in (0, R // 2)
    ]
    return jnp.concatenate(halves, axis=0)


# speedup vs baseline: 1.0588x; 1.0588x over previous
import functools
import math

import jax
import jax.numpy as jnp
from jax import lax
from jax.experimental import pallas as pl
from jax.experimental.pallas import tpu as pltpu
from jax.experimental.pallas import tpu_sc as plsc

D_EMBED = 64
SCALE = math.sqrt(D_EMBED)

NC, NS = 2, 16
NW = NC * NS
XRC = 4


def _make_kernel(R, S, RK, r0):
    assert RK % (NW * XRC) == 0
    xr_per_w = RK // NW
    n_chunks = xr_per_w // XRC
    assert n_chunks % 2 == 0
    cs = XRC * S
    b_per_w = xr_per_w * S
    splits = []
    o = 0
    while o < cs:
        n = min(128, cs - o)
        splits.append((o, n))
        o += n
    mesh = plsc.VectorSubcoreMesh(
        core_axis_name="c", subcore_axis_name="s",
        num_cores=NC, num_subcores=NS)

    @functools.partial(
        pl.kernel,
        out_type=jax.ShapeDtypeStruct((RK, S, D_EMBED), jnp.float32),
        mesh=mesh,
        scratch_types=[
            pltpu.VMEM((b_per_w,), jnp.int32),
            pltpu.VMEM((cs, D_EMBED), jnp.float32),
            pltpu.VMEM((cs, D_EMBED), jnp.float32),
            pltpu.VMEM((XRC, S, D_EMBED), jnp.float32),
            pltpu.VMEM((XRC, S, D_EMBED), jnp.float32),
            pltpu.SemaphoreType.DMA((2,)),
            pltpu.SemaphoreType.DMA((2,)),
        ],
        compiler_params=pltpu.CompilerParams(use_tc_tiling_on_sc=False),
    )
    def k(x_hbm, table_hbm, out_hbm, idx_v, gb0, gb1, ob0, ob1, gsem, ssem):
        wid = lax.axis_index("s") * NC + lax.axis_index("c")
        pltpu.sync_copy(
            x_hbm.at[pl.ds(r0 * S + wid * b_per_w, b_per_w)], idx_v)
        gbufs = (gb0, gb1)
        obufs = (ob0, ob1)

        def gather_start(g, b):
            for (o, n) in splits:
                pltpu.async_copy(
                    table_hbm.at[idx_v.at[pl.ds(g * cs + o, n)]],
                    gbufs[b].at[pl.ds(o, n)], gsem.at[b])

        def gather_wait(b):
            for (o, n) in splits:
                pltpu.make_async_copy(
                    table_hbm.at[idx_v.at[pl.ds(o, n)]],
                    gbufs[b].at[pl.ds(o, n)], gsem.at[b]).wait()

        def store_start(g, b):
            pltpu.async_copy(
                obufs[b], out_hbm.at[pl.ds(wid * xr_per_w + XRC * g, XRC)],
                ssem.at[b])

        def store_wait(b):
            pltpu.make_async_copy(
                obufs[b], out_hbm.at[pl.ds(0, XRC)], ssem.at[b]).wait()

        def scale_out(b):
            gb, ob = gbufs[b], obufs[b]

            def body_s(s, c):
                def body_r(r, c2):
                    li = r * S + s
                    for p in range(D_EMBED // 16):
                        ob[r, s, pl.ds(p * 16, 16)] = (
                            gb[li, pl.ds(p * 16, 16)] * SCALE)
                    return c2
                return lax.fori_loop(0, XRC, body_r, c)
            lax.fori_loop(0, S, body_s, 0)

        gather_start(0, 0)

        def pair(ti, c):
            for ph in range(2):
                g = 2 * ti + ph
                b, nb = ph, 1 - ph

                @pl.when(jnp.logical_and(g >= 1, g + 1 < n_chunks))
                def _():
                    store_wait(nb)

                @pl.when(g + 1 < n_chunks)
                def _():
                    gather_start(g + 1, nb)

                gather_wait(b)
                scale_out(b)
                store_start(g, b)
            return c
        lax.fori_loop(0, n_chunks // 2, pair, 0)
        store_wait(0)
        store_wait(1)

    return k


def kernel(x, table):
    R, S = x.shape
    x1d = x.reshape(R * S).astype(jnp.int32)
    return _make_kernel(R, S, R, 0)(x1d, table)


# speedup vs baseline: 1.0768x; 1.0169x over previous
import functools
import math

import jax
import jax.numpy as jnp
from jax import lax
from jax.experimental import pallas as pl
from jax.experimental.pallas import tpu as pltpu
from jax.experimental.pallas import tpu_sc as plsc

D_EMBED = 64
SCALE = math.sqrt(D_EMBED)

NC, NS = 2, 16
NW = NC * NS
XRC = 8


def _make_kernel(R, S, RK, r0):
    assert RK % (NW * XRC) == 0
    xr_per_w = RK // NW
    n_chunks = xr_per_w // XRC
    assert n_chunks % 2 == 0
    cs = XRC * S
    b_per_w = xr_per_w * S
    splits = []
    o = 0
    while o < cs:
        n = min(128, cs - o)
        splits.append((o, n))
        o += n
    mesh = plsc.VectorSubcoreMesh(
        core_axis_name="c", subcore_axis_name="s",
        num_cores=NC, num_subcores=NS)

    @functools.partial(
        pl.kernel,
        out_type=jax.ShapeDtypeStruct((RK, S, D_EMBED), jnp.float32),
        mesh=mesh,
        scratch_types=[
            pltpu.VMEM((b_per_w,), jnp.int32),
            pltpu.VMEM((cs, D_EMBED), jnp.float32),
            pltpu.VMEM((cs, D_EMBED), jnp.float32),
            pltpu.VMEM((XRC, S, D_EMBED), jnp.float32),
            pltpu.VMEM((XRC, S, D_EMBED), jnp.float32),
            pltpu.SemaphoreType.DMA((2,)),
            pltpu.SemaphoreType.DMA((2,)),
        ],
        compiler_params=pltpu.CompilerParams(use_tc_tiling_on_sc=False),
    )
    def k(x_hbm, table_hbm, out_hbm, idx_v, gb0, gb1, ob0, ob1, gsem, ssem):
        wid = lax.axis_index("s") * NC + lax.axis_index("c")
        pltpu.sync_copy(
            x_hbm.at[pl.ds(r0 * S + wid * b_per_w, b_per_w)], idx_v)
        gbufs = (gb0, gb1)
        obufs = (ob0, ob1)

        def gather_start(g, b):
            for (o, n) in splits:
                pltpu.async_copy(
                    table_hbm.at[idx_v.at[pl.ds(g * cs + o, n)]],
                    gbufs[b].at[pl.ds(o, n)], gsem.at[b])

        def gather_wait(b):
            for (o, n) in splits:
                pltpu.make_async_copy(
                    table_hbm.at[idx_v.at[pl.ds(o, n)]],
                    gbufs[b].at[pl.ds(o, n)], gsem.at[b]).wait()

        def store_start(g, b):
            pltpu.async_copy(
                obufs[b], out_hbm.at[pl.ds(wid * xr_per_w + XRC * g, XRC)],
                ssem.at[b])

        def store_wait(b):
            pltpu.make_async_copy(
                obufs[b], out_hbm.at[pl.ds(0, XRC)], ssem.at[b]).wait()

        def scale_out(b):
            gb, ob = gbufs[b], obufs[b]

            def body_s(s, c):
                def body_r(r, c2):
                    li = r * S + s
                    for p in range(D_EMBED // 16):
                        ob[r, s, pl.ds(p * 16, 16)] = (
                            gb[li, pl.ds(p * 16, 16)] * SCALE)
                    return c2
                return lax.fori_loop(0, XRC, body_r, c)
            lax.fori_loop(0, S, body_s, 0)

        gather_start(0, 0)

        def pair(ti, c):
            for ph in range(2):
                g = 2 * ti + ph
                b, nb = ph, 1 - ph

                @pl.when(jnp.logical_and(g >= 1, g + 1 < n_chunks))
                def _():
                    store_wait(nb)

                @pl.when(g + 1 < n_chunks)
                def _():
                    gather_start(g + 1, nb)

                gather_wait(b)
                scale_out(b)
                store_start(g, b)
            return c
        lax.fori_loop(0, n_chunks // 2, pair, 0)
        store_wait(0)
        store_wait(1)

    return k


def kernel(x, table):
    R, S = x.shape
    x1d = x.reshape(R * S).astype(jnp.int32)
    return _make_kernel(R, S, R, 0)(x1d, table)
